# baseline (speedup 1.0000x reference)
import functools

import jax
import jax.numpy as jnp
from jax import lax
from jax.experimental import pallas as pl
from jax.experimental.pallas import tpu as pltpu
from jax.experimental.pallas import tpu_sc as plsc

NUM_EMBEDDINGS = 1000000
FEATURES = 64
BATCH = 16384
HIST = 50

VOCAB_PAD = 1000064

NC = 2
NS = 16
NW = NC * NS
TOTAL = BATCH * HIST
CH = 128
N_CHUNKS = TOTAL // CH
CPW = N_CHUNKS // NW
B_CHUNKS = BATCH // CH

RING = 5
TB = 2

_mesh = plsc.VectorSubcoreMesh(core_axis_name="c", subcore_axis_name="s",
                               num_cores=NC, num_subcores=NS)


@functools.partial(
    pl.kernel,
    out_type=jax.ShapeDtypeStruct((HIST, FEATURES, BATCH), jnp.float32),
    mesh=_mesh,
    scratch_types=[
        pltpu.VMEM((CPW, CH), jnp.int32),
        pltpu.VMEM((RING, CH, 128), jnp.float32),
        pltpu.VMEM((TB, FEATURES, CH), jnp.float32),
        pltpu.SemaphoreType.DMA((RING,)),
        pltpu.SemaphoreType.DMA((TB,)),
    ],
    compiler_params=pltpu.CompilerParams(use_tc_tiling_on_sc=True,
                                         needs_layout_passes=False),
)
def _gather_t(tbl_hbm, idx_hbm, out_hbm, idx_v, ring_v, tbuf_v, gsem, wsem):
    wid = lax.axis_index("s") * NC + lax.axis_index("c")
    q0 = wid * CPW
    pltpu.sync_copy(idx_hbm.at[wid], idx_v)

    def out_block(k):
        q = q0 + k
        h = q // B_CHUNKS
        b0 = (q % B_CHUNKS) * CH
        return out_hbm.at[h, :, pl.ds(b0, CH)]

    def start_gather(k, b):
        pltpu.async_copy(tbl_hbm.at[idx_v.at[k]], ring_v.at[b], gsem.at[b])

    def wait_gather(k, b):
        pltpu.make_async_copy(tbl_hbm.at[idx_v.at[k]], ring_v.at[b],
                              gsem.at[b]).wait()

    for k in range(RING):
        start_gather(k, k)

    @pl.loop(0, CPW)
    def _chunk(k):
        b = k % RING
        t = k % TB
        wait_gather(k, b)

        @pl.when(k >= TB)
        def _():
            pltpu.make_async_copy(tbuf_v.at[t], out_block(k - TB),
                                  wsem.at[t]).wait()

        rbuf = ring_v.at[b]
        tb = tbuf_v.at[t]
        iota16 = lax.iota(jnp.int32, 16)

        @plsc.parallel_loop(0, CH, unroll=8)
        def _row(row):
            rvec = jnp.full((16,), row, jnp.int32)
            for c0 in range(0, FEATURES, 16):
                x = rbuf[row, pl.ds(c0, 16)]
                plsc.store_scatter(tb, [c0 + iota16, rvec], x)

        @pl.when(k < CPW - RING)
        def _():
            start_gather(k + RING, b)

        pltpu.async_copy(tb, out_block(k), wsem.at[t])

    for k in range(CPW - TB, CPW):
        pltpu.make_async_copy(tbuf_v.at[k % TB], out_block(k),
                              wsem.at[k % TB]).wait()


def kernel(inputs, embedding):
    tbl = jnp.pad(embedding.astype(jnp.float32),
                  ((0, VOCAB_PAD - NUM_EMBEDDINGS), (0, 128 - FEATURES)))
    idx = jnp.transpose(inputs.astype(jnp.int32)).reshape(NW, CPW, CH)
    out = _gather_t(tbl, idx)
    return jnp.transpose(out, (2, 0, 1))


# speedup vs baseline: 1.6244x; 1.6244x over previous
import functools

import jax
import jax.numpy as jnp
from jax import lax
from jax.experimental import pallas as pl
from jax.experimental.pallas import tpu as pltpu
from jax.experimental.pallas import tpu_sc as plsc

NUM_EMBEDDINGS = 1000000
FEATURES = 64
BATCH = 16384
HIST = 50

VOCAB_PAD = 1000064

NC = 2
NS = 16
NW = NC * NS
TOTAL = BATCH * HIST
CH = 128
N_CHUNKS = TOTAL // CH
CPW = N_CHUNKS // NW
B_CHUNKS = BATCH // CH

RING = 5
TB = 2

_mesh = plsc.VectorSubcoreMesh(core_axis_name="c", subcore_axis_name="s",
                               num_cores=NC, num_subcores=NS)


@functools.partial(
    pl.kernel,
    out_type=jax.ShapeDtypeStruct((HIST, FEATURES, BATCH), jnp.float32),
    mesh=_mesh,
    scratch_types=[
        pltpu.VMEM((CPW, CH), jnp.int32),
        pltpu.VMEM((RING, CH, 128), jnp.float32),
        pltpu.VMEM((TB, FEATURES, CH), jnp.float32),
        pltpu.SemaphoreType.DMA((RING,)),
        pltpu.SemaphoreType.DMA((TB,)),
    ],
    compiler_params=pltpu.CompilerParams(use_tc_tiling_on_sc=True,
                                         needs_layout_passes=False),
)
def _gather_t(tbl_hbm, idx_hbm, out_hbm, idx_v, ring_v, tbuf_v, gsem, wsem):
    wid = lax.axis_index("s") * NC + lax.axis_index("c")
    q0 = wid * CPW
    pltpu.sync_copy(idx_hbm.at[wid], idx_v)

    def out_block(k):
        q = q0 + k
        h = q // B_CHUNKS
        b0 = (q % B_CHUNKS) * CH
        return out_hbm.at[h, :, pl.ds(b0, CH)]

    def start_gather(k, b):
        pltpu.async_copy(tbl_hbm.at[idx_v.at[k]], ring_v.at[b], gsem.at[b])

    def wait_gather(k, b):
        pltpu.make_async_copy(tbl_hbm.at[idx_v.at[k]], ring_v.at[b],
                              gsem.at[b]).wait()

    for k in range(RING):
        start_gather(k, k)

    @pl.loop(0, CPW)
    def _chunk(k):
        b = k % RING
        t = k % TB
        wait_gather(k, b)

        @pl.when(k >= TB)
        def _():
            pltpu.make_async_copy(tbuf_v.at[t], out_block(k - TB),
                                  wsem.at[t]).wait()

        rbuf = ring_v.at[b]
        tb = tbuf_v.at[t]
        iota16 = lax.iota(jnp.int32, 16)

        @plsc.parallel_loop(0, CH, step=16, unroll=2)
        def _blk(b0):
            rowidx = b0 + iota16
            for c0 in range(0, FEATURES, 16):
                for kk in range(16):
                    colidx = c0 + ((iota16 + kk) & 15)
                    x = plsc.load_gather(rbuf, [rowidx, colidx])
                    plsc.store_scatter(tb, [colidx, rowidx], x)

        @pl.when(k < CPW - RING)
        def _():
            start_gather(k + RING, b)

        pltpu.async_copy(tb, out_block(k), wsem.at[t])

    for k in range(CPW - TB, CPW):
        pltpu.make_async_copy(tbuf_v.at[k % TB], out_block(k),
                              wsem.at[k % TB]).wait()


def kernel(inputs, embedding):
    tbl = jnp.pad(embedding.astype(jnp.float32),
                  ((0, VOCAB_PAD - NUM_EMBEDDINGS), (0, 128 - FEATURES)))
    idx = jnp.transpose(inputs.astype(jnp.int32)).reshape(NW, CPW, CH)
    out = _gather_t(tbl, idx)
    return jnp.transpose(out, (2, 0, 1))


# speedup vs baseline: 2.2761x; 1.4012x over previous
import functools

import jax
import jax.numpy as jnp
from jax import lax
from jax.experimental import pallas as pl
from jax.experimental.pallas import tpu as pltpu
from jax.experimental.pallas import tpu_sc as plsc

NUM_EMBEDDINGS = 1000000
FEATURES = 64
BATCH = 16384
HIST = 50

VOCAB_PAD = 1000064

NC = 2
NS = 16
NW = NC * NS
TOTAL = BATCH * HIST
CH = 128
N_CHUNKS = TOTAL // CH
CPW = N_CHUNKS // NW
B_CHUNKS = BATCH // CH

RING = 5
TB = 2

_mesh = plsc.VectorSubcoreMesh(core_axis_name="c", subcore_axis_name="s",
                               num_cores=NC, num_subcores=NS)

N_STRIPS = 7812
SPW_LO = N_STRIPS // NW
S_EXTRA = N_STRIPS - SPW_LO * NW


@functools.partial(
    pl.kernel,
    out_type=jax.ShapeDtypeStruct((VOCAB_PAD, 128), jnp.float32),
    mesh=_mesh,
    scratch_types=[
        pltpu.VMEM((2, FEATURES, 128), jnp.float32),
        pltpu.VMEM((2, 128, 128), jnp.float32),
        pltpu.SemaphoreType.DMA((2,)),
        pltpu.SemaphoreType.DMA((2,)),
    ],
    compiler_params=pltpu.CompilerParams(use_tc_tiling_on_sc=True,
                                         needs_layout_passes=False),
)
def _repack(emb_hbm, tail_hbm, out_hbm, sin_v, sout_v, rsem, wsem):
    wid = lax.axis_index("s") * NC + lax.axis_index("c")
    n_s = jnp.where(wid < S_EXTRA, SPW_LO + 1, SPW_LO)
    iota16 = lax.iota(jnp.int32, 16)

    def src(k):
        return emb_hbm.at[:, pl.ds((k * NW + wid) * 128, 128)]

    def dst(k):
        return out_hbm.at[pl.ds((k * NW + wid) * 128, 128), :]

    def start_read(k, b):
        pltpu.async_copy(src(k), sin_v.at[b], rsem.at[b])

    start_read(0, 0)
    start_read(1, 1)

    @pl.loop(0, n_s)
    def _strip(k):
        b = k % 2
        pltpu.make_async_copy(src(k), sin_v.at[b], rsem.at[b]).wait()

        @pl.when(k >= 2)
        def _():
            pltpu.make_async_copy(sout_v.at[b], dst(k - 2), wsem.at[b]).wait()

        sin = sin_v.at[b]
        sout = sout_v.at[b]

        @plsc.parallel_loop(0, 128, step=16, unroll=2)
        def _blk(b0):
            colidx = b0 + iota16
            for c0 in range(0, FEATURES, 16):
                for kk in range(16):
                    rowidx = c0 + ((iota16 + kk) & 15)
                    x = plsc.load_gather(sin, [rowidx, colidx])
                    plsc.store_scatter(sout, [colidx, rowidx], x)

        @pl.when(k + 2 < n_s)
        def _():
            start_read(k + 2, b)

        pltpu.async_copy(sout_v.at[b], dst(k), wsem.at[b])

    @pl.loop(n_s - 2, n_s)
    def _drain(k):
        pltpu.make_async_copy(sout_v.at[k % 2], dst(k), wsem.at[k % 2]).wait()

    @pl.when(wid == 0)
    def _():
        pltpu.sync_copy(tail_hbm, sin_v.at[0])
        pltpu.sync_copy(sin_v.at[0],
                        out_hbm.at[pl.ds(N_STRIPS * 128, FEATURES), :])


@functools.partial(
    pl.kernel,
    out_type=jax.ShapeDtypeStruct((HIST, FEATURES, BATCH), jnp.float32),
    mesh=_mesh,
    scratch_types=[
        pltpu.VMEM((CPW, CH), jnp.int32),
        pltpu.VMEM((RING, CH, 128), jnp.float32),
        pltpu.VMEM((TB, FEATURES, CH), jnp.float32),
        pltpu.SemaphoreType.DMA((RING,)),
        pltpu.SemaphoreType.DMA((TB,)),
    ],
    compiler_params=pltpu.CompilerParams(use_tc_tiling_on_sc=True,
                                         needs_layout_passes=False),
)
def _gather_t(tbl_hbm, idx_hbm, out_hbm, idx_v, ring_v, tbuf_v, gsem, wsem):
    wid = lax.axis_index("s") * NC + lax.axis_index("c")
    q0 = wid * CPW
    pltpu.sync_copy(idx_hbm.at[wid], idx_v)

    def out_block(k):
        q = q0 + k
        h = q // B_CHUNKS
        b0 = (q % B_CHUNKS) * CH
        return out_hbm.at[h, :, pl.ds(b0, CH)]

    def start_gather(k, b):
        pltpu.async_copy(tbl_hbm.at[idx_v.at[k]], ring_v.at[b], gsem.at[b])

    def wait_gather(k, b):
        pltpu.make_async_copy(tbl_hbm.at[idx_v.at[k]], ring_v.at[b],
                              gsem.at[b]).wait()

    for k in range(RING):
        start_gather(k, k)

    @pl.loop(0, CPW)
    def _chunk(k):
        b = k % RING
        t = k % TB
        wait_gather(k, b)

        @pl.when(k >= TB)
        def _():
            pltpu.make_async_copy(tbuf_v.at[t], out_block(k - TB),
                                  wsem.at[t]).wait()

        rbuf = ring_v.at[b]
        tb = tbuf_v.at[t]
        iota16 = lax.iota(jnp.int32, 16)

        @plsc.parallel_loop(0, CH, step=16, unroll=2)
        def _blk(b0):
            rowidx = b0 + iota16
            for c0 in range(0, FEATURES, 16):
                for kk in range(16):
                    colidx = c0 + ((iota16 + kk) & 15)
                    x = plsc.load_gather(rbuf, [rowidx, colidx])
                    plsc.store_scatter(tb, [colidx, rowidx], x)

        @pl.when(k < CPW - RING)
        def _():
            start_gather(k + RING, b)

        pltpu.async_copy(tb, out_block(k), wsem.at[t])

    for k in range(CPW - TB, CPW):
        pltpu.make_async_copy(tbuf_v.at[k % TB], out_block(k),
                              wsem.at[k % TB]).wait()


def kernel(inputs, embedding):
    emb = embedding.astype(jnp.float32)
    emb_t = jnp.transpose(emb)
    tail = jnp.pad(emb[N_STRIPS * 128:], ((0, 0), (0, 128 - FEATURES)))
    tbl = _repack(emb_t, tail)
    idx = jnp.transpose(inputs.astype(jnp.int32)).reshape(NW, CPW, CH)
    out = _gather_t(tbl, idx)
    return jnp.transpose(out, (2, 0, 1))


# speedup vs baseline: 2.2907x; 1.0064x over previous
import functools

import jax
import jax.numpy as jnp
from jax import lax
from jax.experimental import pallas as pl
from jax.experimental.pallas import tpu as pltpu
from jax.experimental.pallas import tpu_sc as plsc

NUM_EMBEDDINGS = 1000000
FEATURES = 64
BATCH = 16384
HIST = 50

VOCAB_PAD = 1000064

NC = 2
NS = 16
NW = NC * NS
TOTAL = BATCH * HIST
CH = 128
N_CHUNKS = TOTAL // CH
CPW = N_CHUNKS // NW
B_CHUNKS = BATCH // CH

RING = 5
TB = 2

_mesh = plsc.VectorSubcoreMesh(core_axis_name="c", subcore_axis_name="s",
                               num_cores=NC, num_subcores=NS)

N_STRIPS = 7812
SPW_LO = N_STRIPS // NW
S_EXTRA = N_STRIPS - SPW_LO * NW


@functools.partial(
    pl.kernel,
    out_type=jax.ShapeDtypeStruct((VOCAB_PAD, 128), jnp.float32),
    mesh=_mesh,
    scratch_types=[
        pltpu.VMEM((3, FEATURES, 128), jnp.float32),
        pltpu.VMEM((3, 128, 128), jnp.float32),
        pltpu.SemaphoreType.DMA((3,)),
        pltpu.SemaphoreType.DMA((3,)),
    ],
    compiler_params=pltpu.CompilerParams(use_tc_tiling_on_sc=True,
                                         needs_layout_passes=False),
)
def _repack(emb_hbm, tail_hbm, out_hbm, sin_v, sout_v, rsem, wsem):
    wid = lax.axis_index("s") * NC + lax.axis_index("c")
    n_s = jnp.where(wid < S_EXTRA, SPW_LO + 1, SPW_LO)
    iota16 = lax.iota(jnp.int32, 16)

    def src(k):
        return emb_hbm.at[:, pl.ds((k * NW + wid) * 128, 128)]

    def dst(k):
        return out_hbm.at[pl.ds((k * NW + wid) * 128, 128), :]

    def start_read(k, b):
        pltpu.async_copy(src(k), sin_v.at[b], rsem.at[b])

    start_read(0, 0)
    start_read(1, 1)
    start_read(2, 2)

    @pl.loop(0, n_s)
    def _strip(k):
        b = k % 3
        pltpu.make_async_copy(src(k), sin_v.at[b], rsem.at[b]).wait()

        @pl.when(k >= 3)
        def _():
            pltpu.make_async_copy(sout_v.at[b], dst(k - 3), wsem.at[b]).wait()

        sin = sin_v.at[b]
        sout = sout_v.at[b]

        @plsc.parallel_loop(0, 128, step=16, unroll=2)
        def _blk(b0):
            colidx = b0 + iota16
            for c0 in range(0, FEATURES, 16):
                for kk in range(16):
                    rowidx = c0 + ((iota16 + kk) & 15)
                    x = plsc.load_gather(sin, [rowidx, colidx])
                    plsc.store_scatter(sout, [colidx, rowidx], x)

        @pl.when(k + 3 < n_s)
        def _():
            start_read(k + 3, b)

        pltpu.async_copy(sout_v.at[b], dst(k), wsem.at[b])

    @pl.loop(n_s - 3, n_s)
    def _drain(k):
        pltpu.make_async_copy(sout_v.at[k % 3], dst(k), wsem.at[k % 3]).wait()

    @pl.when(wid == 0)
    def _():
        pltpu.sync_copy(tail_hbm, sin_v.at[0])
        pltpu.sync_copy(sin_v.at[0],
                        out_hbm.at[pl.ds(N_STRIPS * 128, FEATURES), :])


@functools.partial(
    pl.kernel,
    out_type=jax.ShapeDtypeStruct((HIST, FEATURES, BATCH), jnp.float32),
    mesh=_mesh,
    scratch_types=[
        pltpu.VMEM((CPW, CH), jnp.int32),
        pltpu.VMEM((RING, CH, 128), jnp.float32),
        pltpu.VMEM((TB, FEATURES, CH), jnp.float32),
        pltpu.SemaphoreType.DMA((RING,)),
        pltpu.SemaphoreType.DMA((TB,)),
    ],
    compiler_params=pltpu.CompilerParams(use_tc_tiling_on_sc=True,
                                         needs_layout_passes=False),
)
def _gather_t(tbl_hbm, idx_hbm, out_hbm, idx_v, ring_v, tbuf_v, gsem, wsem):
    wid = lax.axis_index("s") * NC + lax.axis_index("c")
    q0 = wid * CPW
    pltpu.sync_copy(idx_hbm.at[wid], idx_v)

    def out_block(k):
        q = q0 + k
        h = q // B_CHUNKS
        b0 = (q % B_CHUNKS) * CH
        return out_hbm.at[h, :, pl.ds(b0, CH)]

    def start_gather(k, b):
        pltpu.async_copy(tbl_hbm.at[idx_v.at[k]], ring_v.at[b], gsem.at[b])

    def wait_gather(k, b):
        pltpu.make_async_copy(tbl_hbm.at[idx_v.at[k]], ring_v.at[b],
                              gsem.at[b]).wait()

    for k in range(RING):
        start_gather(k, k)

    @pl.loop(0, CPW)
    def _chunk(k):
        b = k % RING
        t = k % TB
        wait_gather(k, b)

        @pl.when(k >= TB)
        def _():
            pltpu.make_async_copy(tbuf_v.at[t], out_block(k - TB),
                                  wsem.at[t]).wait()

        rbuf = ring_v.at[b]
        tb = tbuf_v.at[t]
        iota16 = lax.iota(jnp.int32, 16)

        @plsc.parallel_loop(0, CH, step=16, unroll=2)
        def _blk(b0):
            rowidx = b0 + iota16
            for c0 in range(0, FEATURES, 16):
                for kk in range(16):
                    colidx = c0 + ((iota16 + kk) & 15)
                    x = plsc.load_gather(rbuf, [rowidx, colidx])
                    plsc.store_scatter(tb, [colidx, rowidx], x)

        @pl.when(k < CPW - RING)
        def _():
            start_gather(k + RING, b)

        pltpu.async_copy(tb, out_block(k), wsem.at[t])

    for k in range(CPW - TB, CPW):
        pltpu.make_async_copy(tbuf_v.at[k % TB], out_block(k),
                              wsem.at[k % TB]).wait()


def kernel(inputs, embedding):
    emb = embedding.astype(jnp.float32)
    emb_t = jnp.transpose(emb)
    tail = jnp.pad(emb[N_STRIPS * 128:], ((0, 0), (0, 128 - FEATURES)))
    tbl = _repack(emb_t, tail)
    idx = jnp.transpose(inputs.astype(jnp.int32)).reshape(NW, CPW, CH)
    out = _gather_t(tbl, idx)
    return jnp.transpose(out, (2, 0, 1))


# speedup vs baseline: 2.3199x; 1.0128x over previous
import functools

import jax
import jax.numpy as jnp
from jax import lax
from jax.experimental import pallas as pl
from jax.experimental.pallas import tpu as pltpu
from jax.experimental.pallas import tpu_sc as plsc

NUM_EMBEDDINGS = 1000000
FEATURES = 64
BATCH = 16384
HIST = 50

VOCAB_PAD = 1000064

NC = 2
NS = 16
NW = NC * NS
TOTAL = BATCH * HIST
CH = 128
N_CHUNKS = TOTAL // CH
CPW = N_CHUNKS // NW
B_CHUNKS = BATCH // CH

RING = 5
TB = 2

_mesh = plsc.VectorSubcoreMesh(core_axis_name="c", subcore_axis_name="s",
                               num_cores=NC, num_subcores=NS)

STRIP_W = 256
N_STRIPS = 999936 // STRIP_W
SPW_LO = N_STRIPS // NW
S_EXTRA = N_STRIPS - SPW_LO * NW


@functools.partial(
    pl.kernel,
    out_type=jax.ShapeDtypeStruct((VOCAB_PAD, 128), jnp.float32),
    mesh=_mesh,
    scratch_types=[
        pltpu.VMEM((2, FEATURES, STRIP_W), jnp.float32),
        pltpu.VMEM((2, STRIP_W, 128), jnp.float32),
        pltpu.SemaphoreType.DMA((2,)),
        pltpu.SemaphoreType.DMA((2,)),
    ],
    compiler_params=pltpu.CompilerParams(use_tc_tiling_on_sc=True,
                                         needs_layout_passes=False),
)
def _repack(emb_hbm, tail_hbm, out_hbm, sin_v, sout_v, rsem, wsem):
    wid = lax.axis_index("s") * NC + lax.axis_index("c")
    n_s = jnp.where(wid < S_EXTRA, SPW_LO + 1, SPW_LO)
    iota16 = lax.iota(jnp.int32, 16)

    def src(k):
        return emb_hbm.at[:, pl.ds((k * NW + wid) * STRIP_W, STRIP_W)]

    def dst(k):
        return out_hbm.at[pl.ds((k * NW + wid) * STRIP_W, STRIP_W), :]

    def start_read(k, b):
        pltpu.async_copy(src(k), sin_v.at[b], rsem.at[b])

    start_read(0, 0)
    start_read(1, 1)

    @pl.loop(0, n_s)
    def _strip(k):
        b = k % 2
        pltpu.make_async_copy(src(k), sin_v.at[b], rsem.at[b]).wait()

        @pl.when(k >= 2)
        def _():
            pltpu.make_async_copy(sout_v.at[b], dst(k - 2), wsem.at[b]).wait()

        sin = sin_v.at[b]
        sout = sout_v.at[b]

        @plsc.parallel_loop(0, STRIP_W, step=16, unroll=2)
        def _blk(b0):
            colidx = b0 + iota16
            for c0 in range(0, FEATURES, 16):
                for kk in range(16):
                    rowidx = c0 + ((iota16 + kk) & 15)
                    x = plsc.load_gather(sin, [rowidx, colidx])
                    plsc.store_scatter(sout, [colidx, rowidx], x)

        @pl.when(k + 2 < n_s)
        def _():
            start_read(k + 2, b)

        pltpu.async_copy(sout_v.at[b], dst(k), wsem.at[b])

    @pl.loop(n_s - 2, n_s)
    def _drain(k):
        pltpu.make_async_copy(sout_v.at[k % 2], dst(k), wsem.at[k % 2]).wait()

    @pl.when(wid == 0)
    def _():
        pltpu.sync_copy(tail_hbm, sout_v.at[0, pl.ds(0, FEATURES), :])
        pltpu.sync_copy(sout_v.at[0, pl.ds(0, FEATURES), :],
                        out_hbm.at[pl.ds(N_STRIPS * STRIP_W, FEATURES), :])


@functools.partial(
    pl.kernel,
    out_type=jax.ShapeDtypeStruct((HIST, FEATURES, BATCH), jnp.float32),
    mesh=_mesh,
    scratch_types=[
        pltpu.VMEM((CPW, CH), jnp.int32),
        pltpu.VMEM((RING, CH, 128), jnp.float32),
        pltpu.VMEM((TB, FEATURES, CH), jnp.float32),
        pltpu.SemaphoreType.DMA((RING,)),
        pltpu.SemaphoreType.DMA((TB,)),
    ],
    compiler_params=pltpu.CompilerParams(use_tc_tiling_on_sc=True,
                                         needs_layout_passes=False),
)
def _gather_t(tbl_hbm, idx_hbm, out_hbm, idx_v, ring_v, tbuf_v, gsem, wsem):
    wid = lax.axis_index("s") * NC + lax.axis_index("c")
    q0 = wid * CPW
    pltpu.sync_copy(idx_hbm.at[wid], idx_v)

    def out_block(k):
        q = q0 + k
        h = q // B_CHUNKS
        b0 = (q % B_CHUNKS) * CH
        return out_hbm.at[h, :, pl.ds(b0, CH)]

    def start_gather(k, b):
        pltpu.async_copy(tbl_hbm.at[idx_v.at[k]], ring_v.at[b], gsem.at[b])

    def wait_gather(k, b):
        pltpu.make_async_copy(tbl_hbm.at[idx_v.at[k]], ring_v.at[b],
                              gsem.at[b]).wait()

    for k in range(RING):
        start_gather(k, k)

    @pl.loop(0, CPW)
    def _chunk(k):
        b = k % RING
        t = k % TB
        wait_gather(k, b)

        @pl.when(k >= TB)
        def _():
            pltpu.make_async_copy(tbuf_v.at[t], out_block(k - TB),
                                  wsem.at[t]).wait()

        rbuf = ring_v.at[b]
        tb = tbuf_v.at[t]
        iota16 = lax.iota(jnp.int32, 16)

        @plsc.parallel_loop(0, CH, step=16, unroll=2)
        def _blk(b0):
            rowidx = b0 + iota16
            for c0 in range(0, FEATURES, 16):
                for kk in range(16):
                    colidx = c0 + ((iota16 + kk) & 15)
                    x = plsc.load_gather(rbuf, [rowidx, colidx])
                    plsc.store_scatter(tb, [colidx, rowidx], x)

        @pl.when(k < CPW - RING)
        def _():
            start_gather(k + RING, b)

        pltpu.async_copy(tb, out_block(k), wsem.at[t])

    for k in range(CPW - TB, CPW):
        pltpu.make_async_copy(tbuf_v.at[k % TB], out_block(k),
                              wsem.at[k % TB]).wait()


def kernel(inputs, embedding):
    emb = embedding.astype(jnp.float32)
    emb_t = jnp.transpose(emb)
    tail = jnp.pad(emb[N_STRIPS * STRIP_W:], ((0, 0), (0, 128 - FEATURES)))
    tbl = _repack(emb_t, tail)
    idx = jnp.transpose(inputs.astype(jnp.int32)).reshape(NW, CPW, CH)
    out = _gather_t(tbl, idx)
    return jnp.transpose(out, (2, 0, 1))
